# baseline (speedup 1.0000x reference)
import functools

import jax
import jax.numpy as jnp
from jax import lax
from jax.experimental import pallas as pl
from jax.experimental.pallas import tpu as pltpu
from jax.experimental.pallas import tpu_sc as plsc

N_NODES = 10000
N_EDGES = 320000
D_IN = 128
D_HID = 256
D_OUT = 40

NW = 32
CH = 128
NP = 10240
EP = NW * 80 * CH
EPW = EP // NW
ROWS_PER_SUB = NP // 16

_mesh = plsc.VectorSubcoreMesh(core_axis_name="c", subcore_axis_name="s")


def _fill(ref, rows, cols, value):
    k = cols // 16
    v = jnp.full((16,), value, jnp.float32)

    def body(j, _):
        r = j // k
        c = (j % k) * 16
        ref[r, pl.ds(c, 16)] = v
        return 0

    lax.fori_loop(0, rows * k, body, 0)


CPW = EPW // CH
NB = 4


def _make_sc_degree():
    @functools.partial(
        pl.kernel,
        out_type=jax.ShapeDtypeStruct((2, NP, 16), jnp.float32),
        mesh=_mesh,
        scratch_types=[
            pltpu.VMEM((CPW, CH), jnp.int32),
            pltpu.VMEM((CH, 16), jnp.float32),
            pltpu.VMEM_SHARED((NP, 16), jnp.float32),
            pltpu.SemaphoreType.DMA,
        ],
    )
    def deg_kernel(dst_hbm, out_hbm, dst_all, ones_v, acc_sh, ssem):
        c = lax.axis_index("c")
        s = lax.axis_index("s")
        wid = s * 2 + c
        _fill(ones_v, CH, 16, 0.0)
        for k in range(ROWS_PER_SUB // CH):
            pltpu.sync_copy(ones_v, acc_sh.at[pl.ds(s * ROWS_PER_SUB + k * CH, CH)])
        pltpu.sync_copy(dst_hbm.at[pl.ds(wid * CPW, CPW)], dst_all)
        _fill(ones_v, CH, 16, 1.0)
        plsc.subcore_barrier()

        def body(g, _):
            descs = []
            for b in range(NB):
                t = g * NB + b
                descs.append(pltpu.async_copy(
                    ones_v, acc_sh.at[dst_all.at[t]], ssem, add=True))
            for d in descs:
                d.wait()
            return 0

        lax.fori_loop(0, CPW // NB, body, 0)
        plsc.subcore_barrier()
        pltpu.sync_copy(
            acc_sh.at[pl.ds(s * ROWS_PER_SUB, ROWS_PER_SUB)],
            out_hbm.at[c, pl.ds(s * ROWS_PER_SUB, ROWS_PER_SUB)],
        )

    return deg_kernel


CSB = EP // CH // 16


def _make_sc_agg_colsplit():
    nb = 4

    @functools.partial(
        pl.kernel,
        out_type=jax.ShapeDtypeStruct((2, NP, 64), jnp.float32),
        mesh=_mesh,
        compiler_params=pltpu.CompilerParams(use_tc_tiling_on_sc=False),
        scratch_types=[
            pltpu.VMEM((CSB, CH), jnp.int32),
            pltpu.VMEM((CSB, CH), jnp.int32),
            pltpu.VMEM((nb, CH, 64), jnp.float32),
            pltpu.VMEM_SHARED((NP, 64), jnp.float32),
            pltpu.SemaphoreType.DMA,
            pltpu.SemaphoreType.DMA,
        ],
    )
    def agg_kernel(table_hbm, src_hbm, dst_hbm, out_hbm, src_all, dst_all,
                   rows, acc_sh, gsem, ssem):
        c = lax.axis_index("c")
        s = lax.axis_index("s")
        tbl = table_hbm.at[c]
        _fill(rows.at[0], CH, 64, 0.0)
        for k in range(ROWS_PER_SUB // CH):
            pltpu.sync_copy(rows.at[0], acc_sh.at[pl.ds(s * ROWS_PER_SUB + k * CH, CH)])
        pltpu.sync_copy(src_hbm.at[pl.ds(s * CSB, CSB)], src_all)
        pltpu.sync_copy(dst_hbm.at[pl.ds(s * CSB, CSB)], dst_all)
        plsc.subcore_barrier()

        def body(g, _):
            @pl.when(g > 0)
            def _():
                for b in range(nb):
                    pltpu.make_async_copy(
                        rows.at[b], acc_sh.at[pl.ds(0, CH)], ssem).wait()
            gds = []
            for b in range(nb):
                t = g * nb + b
                gds.append(pltpu.async_copy(
                    tbl.at[src_all.at[t]], rows.at[b], gsem))
            for b in range(nb):
                t = g * nb + b
                gds[b].wait()
                pltpu.async_copy(
                    rows.at[b], acc_sh.at[dst_all.at[t]], ssem, add=True)
            return 0

        lax.fori_loop(0, CSB // nb, body, 0)
        for b in range(nb):
            pltpu.make_async_copy(rows.at[b], acc_sh.at[pl.ds(0, CH)], ssem).wait()
        plsc.subcore_barrier()
        pltpu.sync_copy(
            acc_sh.at[pl.ds(s * ROWS_PER_SUB, ROWS_PER_SUB)],
            out_hbm.at[c, pl.ds(s * ROWS_PER_SUB, ROWS_PER_SUB)],
        )

    return agg_kernel


def _make_sc_agg_edgesplit(d, nb):
    @functools.partial(
        pl.kernel,
        out_type=jax.ShapeDtypeStruct((2, NP, d), jnp.float32),
        mesh=_mesh,
        compiler_params=pltpu.CompilerParams(use_tc_tiling_on_sc=False),
        scratch_types=[
            pltpu.VMEM((CPW, CH), jnp.int32),
            pltpu.VMEM((CPW, CH), jnp.int32),
            pltpu.VMEM((nb, CH, d), jnp.float32),
            pltpu.VMEM_SHARED((NP, d), jnp.float32),
            pltpu.SemaphoreType.DMA,
            pltpu.SemaphoreType.DMA,
        ],
    )
    def agg_kernel(table_hbm, src_hbm, dst_hbm, out_hbm, src_all, dst_all,
                   rows, acc_sh, gsem, ssem):
        c = lax.axis_index("c")
        s = lax.axis_index("s")
        wid = s * 2 + c
        _fill(rows.at[0], CH, d, 0.0)
        for k in range(ROWS_PER_SUB // CH):
            pltpu.sync_copy(rows.at[0], acc_sh.at[pl.ds(s * ROWS_PER_SUB + k * CH, CH)])
        pltpu.sync_copy(src_hbm.at[pl.ds(wid * CPW, CPW)], src_all)
        pltpu.sync_copy(dst_hbm.at[pl.ds(wid * CPW, CPW)], dst_all)
        plsc.subcore_barrier()

        def body(g, _):
            @pl.when(g > 0)
            def _():
                for b in range(nb):
                    pltpu.make_async_copy(
                        rows.at[b], acc_sh.at[pl.ds(0, CH)], ssem).wait()
            gds = []
            for b in range(nb):
                t = g * nb + b
                gds.append(pltpu.async_copy(
                    table_hbm.at[src_all.at[t]], rows.at[b], gsem))
            for b in range(nb):
                t = g * nb + b
                gds[b].wait()
                pltpu.async_copy(
                    rows.at[b], acc_sh.at[dst_all.at[t]], ssem, add=True)
            return 0

        lax.fori_loop(0, CPW // nb, body, 0)
        for b in range(nb):
            pltpu.make_async_copy(rows.at[b], acc_sh.at[pl.ds(0, CH)], ssem).wait()
        plsc.subcore_barrier()
        pltpu.sync_copy(
            acc_sh.at[pl.ds(s * ROWS_PER_SUB, ROWS_PER_SUB)],
            out_hbm.at[c, pl.ds(s * ROWS_PER_SUB, ROWS_PER_SUB)],
        )

    return agg_kernel


_sc_degree = _make_sc_degree()
_sc_agg128 = _make_sc_agg_colsplit()
_sc_agg48 = _make_sc_agg_edgesplit(48, 8)

_TCB = 1024
_GRID = NP // _TCB


def _prep1_body(degp_ref, x_ref, table1_ref, dinvb_ref):
    deg = degp_ref[0, :, 0:1] + degp_ref[1, :, 0:1] + 1.0
    dinv = lax.rsqrt(deg)
    db = jnp.broadcast_to(dinv, (_TCB, D_IN))
    dinvb_ref[...] = db
    t1 = db * x_ref[...]
    table1_ref[0] = t1[:, :64]
    table1_ref[1] = t1[:, 64:]


def _tc_prep1(deg_parts, x_p):
    return pl.pallas_call(
        _prep1_body,
        grid=(_GRID,),
        in_specs=[
            pl.BlockSpec((2, _TCB, 16), lambda i: (0, i, 0)),
            pl.BlockSpec((_TCB, D_IN), lambda i: (i, 0)),
        ],
        out_specs=[
            pl.BlockSpec((2, _TCB, 64), lambda i: (0, i, 0)),
            pl.BlockSpec((_TCB, D_IN), lambda i: (i, 0)),
        ],
        out_shape=[
            jax.ShapeDtypeStruct((2, NP, 64), jnp.float32),
            jax.ShapeDtypeStruct((NP, D_IN), jnp.float32),
        ],
    )(deg_parts, x_p)


def _chain_body(s1p_ref, x_ref, dinvb_ref, W1_ref, b1_ref, W2_ref,
                table2_ref, P_ref):
    db = dinvb_ref[...]
    S1 = jnp.concatenate([s1p_ref[0], s1p_ref[1]], axis=1)
    agg1 = db * S1 + db * db * x_ref[...]
    h1 = jnp.maximum(
        jnp.dot(agg1, W1_ref[...], preferred_element_type=jnp.float32)
        + b1_ref[...], 0.0)
    P = jnp.dot(h1, W2_ref[...], preferred_element_type=jnp.float32)
    P_ref[...] = P
    table2_ref[...] = db[:, :48] * P


def _tc_chain(s1_parts, x_p, dinvb, W1, b1r, W2p):
    return pl.pallas_call(
        _chain_body,
        grid=(_GRID,),
        in_specs=[
            pl.BlockSpec((2, _TCB, 64), lambda i: (0, i, 0)),
            pl.BlockSpec((_TCB, D_IN), lambda i: (i, 0)),
            pl.BlockSpec((_TCB, D_IN), lambda i: (i, 0)),
            pl.BlockSpec((D_IN, D_HID), lambda i: (0, 0)),
            pl.BlockSpec((1, D_HID), lambda i: (0, 0)),
            pl.BlockSpec((D_HID, 48), lambda i: (0, 0)),
        ],
        out_specs=[
            pl.BlockSpec((_TCB, 48), lambda i: (i, 0)),
            pl.BlockSpec((_TCB, 48), lambda i: (i, 0)),
        ],
        out_shape=[
            jax.ShapeDtypeStruct((NP, 48), jnp.float32),
            jax.ShapeDtypeStruct((NP, 48), jnp.float32),
        ],
    )(s1_parts, x_p, dinvb, W1, b1r, W2p)


def _final_body(s2p_ref, P_ref, dinvb_ref, b2_ref, out_ref):
    db = dinvb_ref[:, :48]
    S2 = s2p_ref[0] + s2p_ref[1]
    P = P_ref[...]
    pre = db * S2 + db * db * P + b2_ref[...]
    mask = lax.broadcasted_iota(jnp.int32, (_TCB, 48), 1) < D_OUT
    neg = jnp.full_like(pre, -1e30)
    m = jnp.max(jnp.where(mask, pre, neg), axis=1, keepdims=True)
    e = jnp.where(mask, jnp.exp(pre - m), 0.0)
    ssum = jnp.sum(e, axis=1, keepdims=True)
    out_ref[...] = pre - m - jnp.log(ssum)


def _tc_final(s2_parts, P, dinvb, b2r):
    return pl.pallas_call(
        _final_body,
        grid=(_GRID,),
        in_specs=[
            pl.BlockSpec((2, _TCB, 48), lambda i: (0, i, 0)),
            pl.BlockSpec((_TCB, 48), lambda i: (i, 0)),
            pl.BlockSpec((_TCB, D_IN), lambda i: (i, 0)),
            pl.BlockSpec((1, 48), lambda i: (0, 0)),
        ],
        out_specs=pl.BlockSpec((_TCB, 48), lambda i: (i, 0)),
        out_shape=jax.ShapeDtypeStruct((NP, 48), jnp.float32),
    )(s2_parts, P, dinvb, b2r)


def kernel(x, edge_index, W1, b1, W2, b2):
    src = edge_index[0]
    dst = edge_index[1]
    pad = jnp.full((EP - N_EDGES,), N_NODES, dtype=jnp.int32)
    src_p = jnp.concatenate([src, pad]).reshape(EP // CH, CH)
    dst_p = jnp.concatenate([dst, pad]).reshape(EP // CH, CH)
    x_p = jnp.pad(x, ((0, NP - N_NODES), (0, 0)))
    W2p = jnp.pad(W2, ((0, 0), (0, 48 - D_OUT)))
    b1r = b1.reshape(1, D_HID)
    b2r = jnp.pad(b2, (0, 48 - D_OUT)).reshape(1, 48)

    deg_parts = _sc_degree(dst_p)
    table1, dinvb = _tc_prep1(deg_parts, x_p)
    s1_parts = _sc_agg128(table1, src_p, dst_p)
    table2, P = _tc_chain(s1_parts, x_p, dinvb, W1, b1r, W2p)
    s2_parts = _sc_agg48(table2, src_p, dst_p)
    outp = _tc_final(s2_parts, P, dinvb, b2r)
    return outp[:N_NODES, :D_OUT]


# speedup vs baseline: 1.0140x; 1.0140x over previous
import functools

import jax
import jax.numpy as jnp
from jax import lax
from jax.experimental import pallas as pl
from jax.experimental.pallas import tpu as pltpu
from jax.experimental.pallas import tpu_sc as plsc

N_NODES = 10000
N_EDGES = 320000
D_IN = 128
D_HID = 256
D_OUT = 40

NW = 32
CH = 128
NP = 10240
EP = NW * 80 * CH
EPW = EP // NW
ROWS_PER_SUB = NP // 16

_mesh = plsc.VectorSubcoreMesh(core_axis_name="c", subcore_axis_name="s")


def _fill(ref, rows, cols, value):
    k = cols // 16
    v = jnp.full((16,), value, jnp.float32)

    def body(j, _):
        r = j // k
        c = (j % k) * 16
        ref[r, pl.ds(c, 16)] = v
        return 0

    lax.fori_loop(0, rows * k, body, 0)


CPW = EPW // CH
NB = 4


def _make_sc_degree():
    @functools.partial(
        pl.kernel,
        out_type=jax.ShapeDtypeStruct((2, NP, 16), jnp.float32),
        mesh=_mesh,
        scratch_types=[
            pltpu.VMEM((CPW, CH), jnp.int32),
            pltpu.VMEM((CH, 16), jnp.float32),
            pltpu.VMEM_SHARED((NP, 16), jnp.float32),
            pltpu.SemaphoreType.DMA,
        ],
    )
    def deg_kernel(dst_hbm, out_hbm, dst_all, ones_v, acc_sh, ssem):
        c = lax.axis_index("c")
        s = lax.axis_index("s")
        wid = s * 2 + c
        _fill(ones_v, CH, 16, 0.0)
        for k in range(ROWS_PER_SUB // CH):
            pltpu.sync_copy(ones_v, acc_sh.at[pl.ds(s * ROWS_PER_SUB + k * CH, CH)])
        pltpu.sync_copy(dst_hbm.at[pl.ds(wid * CPW, CPW)], dst_all)
        _fill(ones_v, CH, 16, 1.0)
        plsc.subcore_barrier()

        def body(g, _):
            descs = []
            for b in range(NB):
                t = g * NB + b
                descs.append(pltpu.async_copy(
                    ones_v, acc_sh.at[dst_all.at[t]], ssem, add=True))
            for d in descs:
                d.wait()
            return 0

        lax.fori_loop(0, CPW // NB, body, 0)
        plsc.subcore_barrier()
        pltpu.sync_copy(
            acc_sh.at[pl.ds(s * ROWS_PER_SUB, ROWS_PER_SUB)],
            out_hbm.at[c, pl.ds(s * ROWS_PER_SUB, ROWS_PER_SUB)],
        )

    return deg_kernel


CSB = EP // CH // 16


def _make_sc_agg_colsplit():
    nb = 4
    ng = 4
    tpb = nb * ng

    @functools.partial(
        pl.kernel,
        out_type=jax.ShapeDtypeStruct((2, NP, 64), jnp.float32),
        mesh=_mesh,
        compiler_params=pltpu.CompilerParams(use_tc_tiling_on_sc=False),
        scratch_types=[
            pltpu.VMEM((tpb, 2, CH), jnp.int32),
            pltpu.VMEM((2 * nb, CH, 64), jnp.float32),
            pltpu.VMEM_SHARED((NP, 64), jnp.float32),
            pltpu.SemaphoreType.DMA,
            pltpu.SemaphoreType.DMA,
        ],
    )
    def agg_kernel(table_hbm, ei_hbm, out_hbm, idx_blk, rows, acc_sh,
                   gsem, ssem):
        c = lax.axis_index("c")
        s = lax.axis_index("s")
        tbl = table_hbm.at[c]
        _fill(rows.at[0], CH, 64, 0.0)
        for k in range(ROWS_PER_SUB // CH):
            pltpu.sync_copy(rows.at[0], acc_sh.at[pl.ds(s * ROWS_PER_SUB + k * CH, CH)])
        plsc.subcore_barrier()

        def body(u, _):
            pltpu.sync_copy(ei_hbm.at[pl.ds(s * CSB + u * tpb, tpb)], idx_blk)
            sds = [None] * ng
            for j in range(ng):
                off = (j % 2) * nb
                if j >= 2:
                    for d_ in sds[j - 2]:
                        d_.wait()
                gds = []
                for b in range(nb):
                    gds.append(pltpu.async_copy(
                        tbl.at[idx_blk.at[j * nb + b, 0]], rows.at[off + b],
                        gsem))
                sj = []
                for b in range(nb):
                    gds[b].wait()
                    sj.append(pltpu.async_copy(
                        rows.at[off + b], acc_sh.at[idx_blk.at[j * nb + b, 1]],
                        ssem, add=True))
                sds[j] = sj
            for j in (ng - 2, ng - 1):
                for d_ in sds[j]:
                    d_.wait()
            return 0

        lax.fori_loop(0, CSB // tpb, body, 0)
        plsc.subcore_barrier()
        pltpu.sync_copy(
            acc_sh.at[pl.ds(s * ROWS_PER_SUB, ROWS_PER_SUB)],
            out_hbm.at[c, pl.ds(s * ROWS_PER_SUB, ROWS_PER_SUB)],
        )

    return agg_kernel


def _make_sc_agg_edgesplit(d, nb, ng):
    tpb = nb * ng

    @functools.partial(
        pl.kernel,
        out_type=jax.ShapeDtypeStruct((2, NP, d), jnp.float32),
        mesh=_mesh,
        compiler_params=pltpu.CompilerParams(use_tc_tiling_on_sc=False),
        scratch_types=[
            pltpu.VMEM((CPW, CH), jnp.int32),
            pltpu.VMEM((CPW, CH), jnp.int32),
            pltpu.VMEM((2 * nb, CH, d), jnp.float32),
            pltpu.VMEM_SHARED((NP, d), jnp.float32),
            pltpu.SemaphoreType.DMA,
            pltpu.SemaphoreType.DMA,
        ],
    )
    def agg_kernel(table_hbm, src_hbm, dst_hbm, out_hbm, src_all, dst_all,
                   rows, acc_sh, gsem, ssem):
        c = lax.axis_index("c")
        s = lax.axis_index("s")
        wid = s * 2 + c
        _fill(rows.at[0], CH, d, 0.0)
        for k in range(ROWS_PER_SUB // CH):
            pltpu.sync_copy(rows.at[0], acc_sh.at[pl.ds(s * ROWS_PER_SUB + k * CH, CH)])
        pltpu.sync_copy(src_hbm.at[pl.ds(wid * CPW, CPW)], src_all)
        pltpu.sync_copy(dst_hbm.at[pl.ds(wid * CPW, CPW)], dst_all)
        plsc.subcore_barrier()

        def body(u, _):
            sds = [None] * ng
            for j in range(ng):
                off = (j % 2) * nb
                if j >= 2:
                    for d_ in sds[j - 2]:
                        d_.wait()
                gds = []
                for b in range(nb):
                    t = u * tpb + j * nb + b
                    gds.append(pltpu.async_copy(
                        table_hbm.at[src_all.at[t]], rows.at[off + b], gsem))
                sj = []
                for b in range(nb):
                    t = u * tpb + j * nb + b
                    gds[b].wait()
                    sj.append(pltpu.async_copy(
                        rows.at[off + b], acc_sh.at[dst_all.at[t]],
                        ssem, add=True))
                sds[j] = sj
            for j in (ng - 2, ng - 1):
                for d_ in sds[j]:
                    d_.wait()
            return 0

        lax.fori_loop(0, CPW // tpb, body, 0)
        plsc.subcore_barrier()
        pltpu.sync_copy(
            acc_sh.at[pl.ds(s * ROWS_PER_SUB, ROWS_PER_SUB)],
            out_hbm.at[c, pl.ds(s * ROWS_PER_SUB, ROWS_PER_SUB)],
        )

    return agg_kernel


_sc_degree = _make_sc_degree()
_sc_agg128 = _make_sc_agg_colsplit()
_sc_agg48 = _make_sc_agg_edgesplit(48, 5, 4)

_TCB = 1024
_GRID = NP // _TCB


def _prep1_body(degp_ref, x_ref, table1_ref, dinvb_ref):
    deg = degp_ref[0, :, 0:1] + degp_ref[1, :, 0:1] + 1.0
    dinv = lax.rsqrt(deg)
    db = jnp.broadcast_to(dinv, (_TCB, D_IN))
    dinvb_ref[...] = db
    t1 = db * x_ref[...]
    table1_ref[0] = t1[:, :64]
    table1_ref[1] = t1[:, 64:]


def _tc_prep1(deg_parts, x_p):
    return pl.pallas_call(
        _prep1_body,
        grid=(_GRID,),
        in_specs=[
            pl.BlockSpec((2, _TCB, 16), lambda i: (0, i, 0)),
            pl.BlockSpec((_TCB, D_IN), lambda i: (i, 0)),
        ],
        out_specs=[
            pl.BlockSpec((2, _TCB, 64), lambda i: (0, i, 0)),
            pl.BlockSpec((_TCB, D_IN), lambda i: (i, 0)),
        ],
        out_shape=[
            jax.ShapeDtypeStruct((2, NP, 64), jnp.float32),
            jax.ShapeDtypeStruct((NP, D_IN), jnp.float32),
        ],
    )(deg_parts, x_p)


def _chain_body(s1p_ref, x_ref, dinvb_ref, W1_ref, b1_ref, W2_ref,
                table2_ref, P_ref):
    db = dinvb_ref[...]
    S1 = jnp.concatenate([s1p_ref[0], s1p_ref[1]], axis=1)
    agg1 = db * S1 + db * db * x_ref[...]
    h1 = jnp.maximum(
        jnp.dot(agg1, W1_ref[...], preferred_element_type=jnp.float32)
        + b1_ref[...], 0.0)
    P = jnp.dot(h1, W2_ref[...], preferred_element_type=jnp.float32)
    P_ref[...] = P
    table2_ref[...] = db[:, :48] * P


def _tc_chain(s1_parts, x_p, dinvb, W1, b1r, W2p):
    return pl.pallas_call(
        _chain_body,
        grid=(_GRID,),
        in_specs=[
            pl.BlockSpec((2, _TCB, 64), lambda i: (0, i, 0)),
            pl.BlockSpec((_TCB, D_IN), lambda i: (i, 0)),
            pl.BlockSpec((_TCB, D_IN), lambda i: (i, 0)),
            pl.BlockSpec((D_IN, D_HID), lambda i: (0, 0)),
            pl.BlockSpec((1, D_HID), lambda i: (0, 0)),
            pl.BlockSpec((D_HID, 48), lambda i: (0, 0)),
        ],
        out_specs=[
            pl.BlockSpec((_TCB, 48), lambda i: (i, 0)),
            pl.BlockSpec((_TCB, 48), lambda i: (i, 0)),
        ],
        out_shape=[
            jax.ShapeDtypeStruct((NP, 48), jnp.float32),
            jax.ShapeDtypeStruct((NP, 48), jnp.float32),
        ],
    )(s1_parts, x_p, dinvb, W1, b1r, W2p)


def _final_body(s2p_ref, P_ref, dinvb_ref, b2_ref, out_ref):
    db = dinvb_ref[:, :48]
    S2 = s2p_ref[0] + s2p_ref[1]
    P = P_ref[...]
    pre = db * S2 + db * db * P + b2_ref[...]
    mask = lax.broadcasted_iota(jnp.int32, (_TCB, 48), 1) < D_OUT
    neg = jnp.full_like(pre, -1e30)
    m = jnp.max(jnp.where(mask, pre, neg), axis=1, keepdims=True)
    e = jnp.where(mask, jnp.exp(pre - m), 0.0)
    ssum = jnp.sum(e, axis=1, keepdims=True)
    out_ref[...] = pre - m - jnp.log(ssum)


def _tc_final(s2_parts, P, dinvb, b2r):
    return pl.pallas_call(
        _final_body,
        grid=(_GRID,),
        in_specs=[
            pl.BlockSpec((2, _TCB, 48), lambda i: (0, i, 0)),
            pl.BlockSpec((_TCB, 48), lambda i: (i, 0)),
            pl.BlockSpec((_TCB, D_IN), lambda i: (i, 0)),
            pl.BlockSpec((1, 48), lambda i: (0, 0)),
        ],
        out_specs=pl.BlockSpec((_TCB, 48), lambda i: (i, 0)),
        out_shape=jax.ShapeDtypeStruct((NP, 48), jnp.float32),
    )(s2_parts, P, dinvb, b2r)


def kernel(x, edge_index, W1, b1, W2, b2):
    src = edge_index[0]
    dst = edge_index[1]
    pad = jnp.full((EP - N_EDGES,), N_NODES, dtype=jnp.int32)
    src_p = jnp.concatenate([src, pad]).reshape(EP // CH, CH)
    dst_p = jnp.concatenate([dst, pad]).reshape(EP // CH, CH)
    x_p = jnp.pad(x, ((0, NP - N_NODES), (0, 0)))
    W2p = jnp.pad(W2, ((0, 0), (0, 48 - D_OUT)))
    b1r = b1.reshape(1, D_HID)
    b2r = jnp.pad(b2, (0, 48 - D_OUT)).reshape(1, 48)

    ei_packed = jnp.stack([src_p, dst_p], axis=1)

    deg_parts = _sc_degree(dst_p)
    table1, dinvb = _tc_prep1(deg_parts, x_p)
    s1_parts = _sc_agg128(table1, ei_packed)
    table2, P = _tc_chain(s1_parts, x_p, dinvb, W1, b1r, W2p)
    s2_parts = _sc_agg48(table2, src_p, dst_p)
    outp = _tc_final(s2_parts, P, dinvb, b2r)
    return outp[:N_NODES, :D_OUT]


# speedup vs baseline: 1.3681x; 1.3492x over previous
import functools

import jax
import jax.numpy as jnp
from jax import lax
from jax.experimental import pallas as pl
from jax.experimental.pallas import tpu as pltpu
from jax.experimental.pallas import tpu_sc as plsc

N_NODES = 10000
N_EDGES = 320000
D_IN = 128
D_HID = 256
D_OUT = 40

NW = 32
CH = 128
NP = 10240
EP = NW * 80 * CH
EPW = EP // NW
ROWS_PER_SUB = NP // 16

_mesh = plsc.VectorSubcoreMesh(core_axis_name="c", subcore_axis_name="s")


def _fill(ref, rows, cols, value):
    k = cols // 16
    v = jnp.full((16,), value, jnp.float32)

    def body(j, _):
        r = j // k
        c = (j % k) * 16
        ref[r, pl.ds(c, 16)] = v
        return 0

    lax.fori_loop(0, rows * k, body, 0)


CPW = EPW // CH
NB = 4


def _make_sc_degree():
    @functools.partial(
        pl.kernel,
        out_type=jax.ShapeDtypeStruct((2, NP, 16), jnp.float32),
        mesh=_mesh,
        scratch_types=[
            pltpu.VMEM((CPW, CH), jnp.int32),
            pltpu.VMEM((CH, 16), jnp.float32),
            pltpu.VMEM_SHARED((NP, 16), jnp.float32),
            pltpu.SemaphoreType.DMA,
        ],
    )
    def deg_kernel(dst_hbm, out_hbm, dst_all, ones_v, acc_sh, ssem):
        c = lax.axis_index("c")
        s = lax.axis_index("s")
        wid = s * 2 + c
        _fill(ones_v, CH, 16, 0.0)
        for k in range(ROWS_PER_SUB // CH):
            pltpu.sync_copy(ones_v, acc_sh.at[pl.ds(s * ROWS_PER_SUB + k * CH, CH)])
        pltpu.sync_copy(dst_hbm.at[pl.ds(wid * CPW, CPW)], dst_all)
        _fill(ones_v, CH, 16, 1.0)
        plsc.subcore_barrier()

        def body(g, _):
            descs = []
            for b in range(NB):
                t = g * NB + b
                descs.append(pltpu.async_copy(
                    ones_v, acc_sh.at[dst_all.at[t]], ssem, add=True))
            for d in descs:
                d.wait()
            return 0

        lax.fori_loop(0, CPW // NB, body, 0)
        plsc.subcore_barrier()
        pltpu.sync_copy(
            acc_sh.at[pl.ds(s * ROWS_PER_SUB, ROWS_PER_SUB)],
            out_hbm.at[c, pl.ds(s * ROWS_PER_SUB, ROWS_PER_SUB)],
        )

    return deg_kernel


CSB = EP // CH // 16


def _make_sc_agg_colsplit():
    nb = 2
    ng = 4
    tpb = nb * ng

    @functools.partial(
        pl.kernel,
        out_type=jax.ShapeDtypeStruct((2, NP, 64), jnp.float32),
        mesh=_mesh,
        compiler_params=pltpu.CompilerParams(use_tc_tiling_on_sc=False),
        scratch_types=[
            pltpu.VMEM((tpb, 2, CH), jnp.int32),
            pltpu.VMEM((2 * nb, CH, 64), jnp.float32),
            pltpu.VMEM_SHARED((NP, 64), jnp.float32),
            pltpu.VMEM_SHARED((NP, 64), jnp.float32),
            pltpu.SemaphoreType.DMA,
            pltpu.SemaphoreType.DMA,
        ],
    )
    def agg_kernel(table_hbm, ei_hbm, out_hbm, idx_blk, rows, acc_sh,
                   tbl_sh, gsem, ssem):
        c = lax.axis_index("c")
        s = lax.axis_index("s")
        tbl = tbl_sh
        pltpu.sync_copy(
            table_hbm.at[c, pl.ds(s * ROWS_PER_SUB, ROWS_PER_SUB)],
            tbl_sh.at[pl.ds(s * ROWS_PER_SUB, ROWS_PER_SUB)])
        _fill(rows.at[0], CH, 64, 0.0)
        for k in range(ROWS_PER_SUB // CH):
            pltpu.sync_copy(rows.at[0], acc_sh.at[pl.ds(s * ROWS_PER_SUB + k * CH, CH)])
        plsc.subcore_barrier()

        def body(u, _):
            pltpu.sync_copy(ei_hbm.at[pl.ds(s * CSB + u * tpb, tpb)], idx_blk)
            sds = [None] * ng
            for j in range(ng):
                off = (j % 2) * nb
                if j >= 2:
                    for d_ in sds[j - 2]:
                        d_.wait()
                gds = []
                for b in range(nb):
                    gds.append(pltpu.async_copy(
                        tbl.at[idx_blk.at[j * nb + b, 0]], rows.at[off + b],
                        gsem))
                sj = []
                for b in range(nb):
                    gds[b].wait()
                    sj.append(pltpu.async_copy(
                        rows.at[off + b], acc_sh.at[idx_blk.at[j * nb + b, 1]],
                        ssem, add=True))
                sds[j] = sj
            for j in (ng - 2, ng - 1):
                for d_ in sds[j]:
                    d_.wait()
            return 0

        lax.fori_loop(0, CSB // tpb, body, 0)
        plsc.subcore_barrier()
        pltpu.sync_copy(
            acc_sh.at[pl.ds(s * ROWS_PER_SUB, ROWS_PER_SUB)],
            out_hbm.at[c, pl.ds(s * ROWS_PER_SUB, ROWS_PER_SUB)],
        )

    return agg_kernel


def _make_sc_agg_edgesplit(d, nb, ng):
    tpb = nb * ng

    @functools.partial(
        pl.kernel,
        out_type=jax.ShapeDtypeStruct((2, NP, d), jnp.float32),
        mesh=_mesh,
        compiler_params=pltpu.CompilerParams(use_tc_tiling_on_sc=False),
        scratch_types=[
            pltpu.VMEM((CPW, CH), jnp.int32),
            pltpu.VMEM((CPW, CH), jnp.int32),
            pltpu.VMEM((2 * nb, CH, d), jnp.float32),
            pltpu.VMEM_SHARED((NP, d), jnp.float32),
            pltpu.SemaphoreType.DMA,
            pltpu.SemaphoreType.DMA,
        ],
    )
    def agg_kernel(table_hbm, src_hbm, dst_hbm, out_hbm, src_all, dst_all,
                   rows, acc_sh, gsem, ssem):
        c = lax.axis_index("c")
        s = lax.axis_index("s")
        wid = s * 2 + c
        _fill(rows.at[0], CH, d, 0.0)
        for k in range(ROWS_PER_SUB // CH):
            pltpu.sync_copy(rows.at[0], acc_sh.at[pl.ds(s * ROWS_PER_SUB + k * CH, CH)])
        pltpu.sync_copy(src_hbm.at[pl.ds(wid * CPW, CPW)], src_all)
        pltpu.sync_copy(dst_hbm.at[pl.ds(wid * CPW, CPW)], dst_all)
        plsc.subcore_barrier()

        def body(u, _):
            sds = [None] * ng
            for j in range(ng):
                off = (j % 2) * nb
                if j >= 2:
                    for d_ in sds[j - 2]:
                        d_.wait()
                gds = []
                for b in range(nb):
                    t = u * tpb + j * nb + b
                    gds.append(pltpu.async_copy(
                        table_hbm.at[src_all.at[t]], rows.at[off + b], gsem))
                sj = []
                for b in range(nb):
                    t = u * tpb + j * nb + b
                    gds[b].wait()
                    sj.append(pltpu.async_copy(
                        rows.at[off + b], acc_sh.at[dst_all.at[t]],
                        ssem, add=True))
                sds[j] = sj
            for j in (ng - 2, ng - 1):
                for d_ in sds[j]:
                    d_.wait()
            return 0

        lax.fori_loop(0, CPW // tpb, body, 0)
        plsc.subcore_barrier()
        pltpu.sync_copy(
            acc_sh.at[pl.ds(s * ROWS_PER_SUB, ROWS_PER_SUB)],
            out_hbm.at[c, pl.ds(s * ROWS_PER_SUB, ROWS_PER_SUB)],
        )

    return agg_kernel


_sc_degree = _make_sc_degree()
_sc_agg128 = _make_sc_agg_colsplit()
_sc_agg48 = _make_sc_agg_edgesplit(48, 5, 4)

_TCB = 1024
_GRID = NP // _TCB


def _prep1_body(degp_ref, x_ref, table1_ref, dinvb_ref):
    deg = degp_ref[0, :, 0:1] + degp_ref[1, :, 0:1] + 1.0
    dinv = lax.rsqrt(deg)
    db = jnp.broadcast_to(dinv, (_TCB, D_IN))
    dinvb_ref[...] = db
    t1 = db * x_ref[...]
    table1_ref[0] = t1[:, :64]
    table1_ref[1] = t1[:, 64:]


def _tc_prep1(deg_parts, x_p):
    return pl.pallas_call(
        _prep1_body,
        grid=(_GRID,),
        in_specs=[
            pl.BlockSpec((2, _TCB, 16), lambda i: (0, i, 0)),
            pl.BlockSpec((_TCB, D_IN), lambda i: (i, 0)),
        ],
        out_specs=[
            pl.BlockSpec((2, _TCB, 64), lambda i: (0, i, 0)),
            pl.BlockSpec((_TCB, D_IN), lambda i: (i, 0)),
        ],
        out_shape=[
            jax.ShapeDtypeStruct((2, NP, 64), jnp.float32),
            jax.ShapeDtypeStruct((NP, D_IN), jnp.float32),
        ],
    )(deg_parts, x_p)


def _chain_body(s1p_ref, x_ref, dinvb_ref, W1_ref, b1_ref, W2_ref,
                table2_ref, P_ref):
    db = dinvb_ref[...]
    S1 = jnp.concatenate([s1p_ref[0], s1p_ref[1]], axis=1)
    agg1 = db * S1 + db * db * x_ref[...]
    h1 = jnp.maximum(
        jnp.dot(agg1, W1_ref[...], preferred_element_type=jnp.float32)
        + b1_ref[...], 0.0)
    P = jnp.dot(h1, W2_ref[...], preferred_element_type=jnp.float32)
    P_ref[...] = P
    table2_ref[...] = db[:, :48] * P


def _tc_chain(s1_parts, x_p, dinvb, W1, b1r, W2p):
    return pl.pallas_call(
        _chain_body,
        grid=(_GRID,),
        in_specs=[
            pl.BlockSpec((2, _TCB, 64), lambda i: (0, i, 0)),
            pl.BlockSpec((_TCB, D_IN), lambda i: (i, 0)),
            pl.BlockSpec((_TCB, D_IN), lambda i: (i, 0)),
            pl.BlockSpec((D_IN, D_HID), lambda i: (0, 0)),
            pl.BlockSpec((1, D_HID), lambda i: (0, 0)),
            pl.BlockSpec((D_HID, 48), lambda i: (0, 0)),
        ],
        out_specs=[
            pl.BlockSpec((_TCB, 48), lambda i: (i, 0)),
            pl.BlockSpec((_TCB, 48), lambda i: (i, 0)),
        ],
        out_shape=[
            jax.ShapeDtypeStruct((NP, 48), jnp.float32),
            jax.ShapeDtypeStruct((NP, 48), jnp.float32),
        ],
    )(s1_parts, x_p, dinvb, W1, b1r, W2p)


def _final_body(s2p_ref, P_ref, dinvb_ref, b2_ref, out_ref):
    db = dinvb_ref[:, :48]
    S2 = s2p_ref[0] + s2p_ref[1]
    P = P_ref[...]
    pre = db * S2 + db * db * P + b2_ref[...]
    mask = lax.broadcasted_iota(jnp.int32, (_TCB, 48), 1) < D_OUT
    neg = jnp.full_like(pre, -1e30)
    m = jnp.max(jnp.where(mask, pre, neg), axis=1, keepdims=True)
    e = jnp.where(mask, jnp.exp(pre - m), 0.0)
    ssum = jnp.sum(e, axis=1, keepdims=True)
    out_ref[...] = pre - m - jnp.log(ssum)


def _tc_final(s2_parts, P, dinvb, b2r):
    return pl.pallas_call(
        _final_body,
        grid=(_GRID,),
        in_specs=[
            pl.BlockSpec((2, _TCB, 48), lambda i: (0, i, 0)),
            pl.BlockSpec((_TCB, 48), lambda i: (i, 0)),
            pl.BlockSpec((_TCB, D_IN), lambda i: (i, 0)),
            pl.BlockSpec((1, 48), lambda i: (0, 0)),
        ],
        out_specs=pl.BlockSpec((_TCB, 48), lambda i: (i, 0)),
        out_shape=jax.ShapeDtypeStruct((NP, 48), jnp.float32),
    )(s2_parts, P, dinvb, b2r)


def kernel(x, edge_index, W1, b1, W2, b2):
    src = edge_index[0]
    dst = edge_index[1]
    pad = jnp.full((EP - N_EDGES,), N_NODES, dtype=jnp.int32)
    src_p = jnp.concatenate([src, pad]).reshape(EP // CH, CH)
    dst_p = jnp.concatenate([dst, pad]).reshape(EP // CH, CH)
    x_p = jnp.pad(x, ((0, NP - N_NODES), (0, 0)))
    W2p = jnp.pad(W2, ((0, 0), (0, 48 - D_OUT)))
    b1r = b1.reshape(1, D_HID)
    b2r = jnp.pad(b2, (0, 48 - D_OUT)).reshape(1, 48)

    ei_packed = jnp.stack([src_p, dst_p], axis=1)

    deg_parts = _sc_degree(dst_p)
    table1, dinvb = _tc_prep1(deg_parts, x_p)
    s1_parts = _sc_agg128(table1, ei_packed)
    table2, P = _tc_chain(s1_parts, x_p, dinvb, W1, b1r, W2p)
    s2_parts = _sc_agg48(table2, src_p, dst_p)
    outp = _tc_final(s2_parts, P, dinvb, b2r)
    return outp[:N_NODES, :D_OUT]


# speedup vs baseline: 1.8892x; 1.3809x over previous
import functools

import jax
import jax.numpy as jnp
from jax import lax
from jax.experimental import pallas as pl
from jax.experimental.pallas import tpu as pltpu
from jax.experimental.pallas import tpu_sc as plsc

N_NODES = 10000
N_EDGES = 320000
D_IN = 128
D_HID = 256
D_OUT = 40

NW = 32
CH = 128
NP = 10240
EP = NW * 80 * CH
EPW = EP // NW
ROWS_PER_SUB = NP // 16

_mesh = plsc.VectorSubcoreMesh(core_axis_name="c", subcore_axis_name="s")


def _fill(ref, rows, cols, value):
    k = cols // 16
    v = jnp.full((16,), value, jnp.float32)

    def body(j, _):
        r = j // k
        c = (j % k) * 16
        ref[r, pl.ds(c, 16)] = v
        return 0

    lax.fori_loop(0, rows * k, body, 0)


CPW = EPW // CH
NB = 4


def _make_sc_degree():
    @functools.partial(
        pl.kernel,
        out_type=jax.ShapeDtypeStruct((2, NP, 16), jnp.float32),
        mesh=_mesh,
        scratch_types=[
            pltpu.VMEM((CPW, CH), jnp.int32),
            pltpu.VMEM((CH, 16), jnp.float32),
            pltpu.VMEM_SHARED((NP, 16), jnp.float32),
            pltpu.SemaphoreType.DMA,
        ],
    )
    def deg_kernel(dst_hbm, out_hbm, dst_all, ones_v, acc_sh, ssem):
        c = lax.axis_index("c")
        s = lax.axis_index("s")
        wid = s * 2 + c
        _fill(ones_v, CH, 16, 0.0)
        for k in range(ROWS_PER_SUB // CH):
            pltpu.sync_copy(ones_v, acc_sh.at[pl.ds(s * ROWS_PER_SUB + k * CH, CH)])
        pltpu.sync_copy(dst_hbm.at[pl.ds(wid * CPW, CPW)], dst_all)
        _fill(ones_v, CH, 16, 1.0)
        plsc.subcore_barrier()

        def body(g, _):
            descs = []
            for b in range(NB):
                t = g * NB + b
                descs.append(pltpu.async_copy(
                    ones_v, acc_sh.at[dst_all.at[t]], ssem, add=True))
            for d in descs:
                d.wait()
            return 0

        lax.fori_loop(0, CPW // NB, body, 0)
        plsc.subcore_barrier()
        pltpu.sync_copy(
            acc_sh.at[pl.ds(s * ROWS_PER_SUB, ROWS_PER_SUB)],
            out_hbm.at[c, pl.ds(s * ROWS_PER_SUB, ROWS_PER_SUB)],
        )

    return deg_kernel


CSB = EP // CH // 16


def _make_sc_agg_colsplit():
    nb = 2
    ng = 4
    tpb = nb * ng

    @functools.partial(
        pl.kernel,
        out_type=jax.ShapeDtypeStruct((2, NP, 64), jnp.float32),
        mesh=_mesh,
        compiler_params=pltpu.CompilerParams(use_tc_tiling_on_sc=False),
        scratch_types=[
            pltpu.VMEM((tpb, 2, CH), jnp.int32),
            pltpu.VMEM((2 * nb, CH, 64), jnp.float32),
            pltpu.VMEM_SHARED((NP, 64), jnp.float32),
            pltpu.VMEM_SHARED((NP, 64), jnp.float32),
            pltpu.SemaphoreType.DMA,
            pltpu.SemaphoreType.DMA,
        ],
    )
    def agg_kernel(table_hbm, ei_hbm, out_hbm, idx_blk, rows, acc_sh,
                   tbl_sh, gsem, ssem):
        c = lax.axis_index("c")
        s = lax.axis_index("s")
        tbl = tbl_sh
        pltpu.sync_copy(
            table_hbm.at[c, pl.ds(s * ROWS_PER_SUB, ROWS_PER_SUB)],
            tbl_sh.at[pl.ds(s * ROWS_PER_SUB, ROWS_PER_SUB)])
        _fill(rows.at[0], CH, 64, 0.0)
        for k in range(ROWS_PER_SUB // CH):
            pltpu.sync_copy(rows.at[0], acc_sh.at[pl.ds(s * ROWS_PER_SUB + k * CH, CH)])
        plsc.subcore_barrier()

        def body(u, _):
            pltpu.sync_copy(ei_hbm.at[pl.ds(s * CSB + u * tpb, tpb)], idx_blk)
            sds = [None] * ng
            for j in range(ng):
                off = (j % 2) * nb
                if j >= 2:
                    for d_ in sds[j - 2]:
                        d_.wait()
                gds = []
                for b in range(nb):
                    gds.append(pltpu.async_copy(
                        tbl.at[idx_blk.at[j * nb + b, 0]], rows.at[off + b],
                        gsem))
                sj = []
                for b in range(nb):
                    gds[b].wait()
                    sj.append(pltpu.async_copy(
                        rows.at[off + b], acc_sh.at[idx_blk.at[j * nb + b, 1]],
                        ssem, add=True))
                sds[j] = sj
            for j in (ng - 2, ng - 1):
                for d_ in sds[j]:
                    d_.wait()
            return 0

        lax.fori_loop(0, CSB // tpb, body, 0)
        plsc.subcore_barrier()
        pltpu.sync_copy(
            acc_sh.at[pl.ds(s * ROWS_PER_SUB, ROWS_PER_SUB)],
            out_hbm.at[c, pl.ds(s * ROWS_PER_SUB, ROWS_PER_SUB)],
        )

    return agg_kernel


def _make_sc_agg_edgesplit(d, nb, ng):
    tpb = nb * ng

    @functools.partial(
        pl.kernel,
        out_type=jax.ShapeDtypeStruct((2, NP, d), jnp.float32),
        mesh=_mesh,
        compiler_params=pltpu.CompilerParams(use_tc_tiling_on_sc=False),
        scratch_types=[
            pltpu.VMEM((tpb, 2, CH), jnp.int32),
            pltpu.VMEM((2 * nb, CH, d), jnp.float32),
            pltpu.VMEM_SHARED((NP, d), jnp.float32),
            pltpu.VMEM_SHARED((NP, d), jnp.float32),
            pltpu.SemaphoreType.DMA,
            pltpu.SemaphoreType.DMA,
        ],
    )
    def agg_kernel(table_hbm, ei_hbm, out_hbm, idx_blk, rows, acc_sh,
                   tbl_sh, gsem, ssem):
        c = lax.axis_index("c")
        s = lax.axis_index("s")
        wid = s * 2 + c
        pltpu.sync_copy(
            table_hbm.at[pl.ds(s * ROWS_PER_SUB, ROWS_PER_SUB)],
            tbl_sh.at[pl.ds(s * ROWS_PER_SUB, ROWS_PER_SUB)])
        _fill(rows.at[0], CH, d, 0.0)
        for k in range(ROWS_PER_SUB // CH):
            pltpu.sync_copy(rows.at[0], acc_sh.at[pl.ds(s * ROWS_PER_SUB + k * CH, CH)])
        plsc.subcore_barrier()

        def body(u, _):
            pltpu.sync_copy(ei_hbm.at[pl.ds(wid * CPW + u * tpb, tpb)], idx_blk)
            sds = [None] * ng
            for j in range(ng):
                off = (j % 2) * nb
                if j >= 2:
                    for d_ in sds[j - 2]:
                        d_.wait()
                gds = []
                for b in range(nb):
                    gds.append(pltpu.async_copy(
                        tbl_sh.at[idx_blk.at[j * nb + b, 0]], rows.at[off + b],
                        gsem))
                sj = []
                for b in range(nb):
                    gds[b].wait()
                    sj.append(pltpu.async_copy(
                        rows.at[off + b], acc_sh.at[idx_blk.at[j * nb + b, 1]],
                        ssem, add=True))
                sds[j] = sj
            for j in (ng - 2, ng - 1):
                for d_ in sds[j]:
                    d_.wait()
            return 0

        lax.fori_loop(0, CPW // tpb, body, 0)
        plsc.subcore_barrier()
        pltpu.sync_copy(
            acc_sh.at[pl.ds(s * ROWS_PER_SUB, ROWS_PER_SUB)],
            out_hbm.at[c, pl.ds(s * ROWS_PER_SUB, ROWS_PER_SUB)],
        )

    return agg_kernel


_sc_degree = _make_sc_degree()
_sc_agg128 = _make_sc_agg_colsplit()
_sc_agg48 = _make_sc_agg_edgesplit(48, 4, 4)

_TCB = 1024
_GRID = NP // _TCB


def _prep1_body(degp_ref, x_ref, table1_ref, dinvb_ref):
    deg = degp_ref[0, :, 0:1] + degp_ref[1, :, 0:1] + 1.0
    dinv = lax.rsqrt(deg)
    db = jnp.broadcast_to(dinv, (_TCB, D_IN))
    dinvb_ref[...] = db
    t1 = db * x_ref[...]
    table1_ref[0] = t1[:, :64]
    table1_ref[1] = t1[:, 64:]


def _tc_prep1(deg_parts, x_p):
    return pl.pallas_call(
        _prep1_body,
        grid=(_GRID,),
        in_specs=[
            pl.BlockSpec((2, _TCB, 16), lambda i: (0, i, 0)),
            pl.BlockSpec((_TCB, D_IN), lambda i: (i, 0)),
        ],
        out_specs=[
            pl.BlockSpec((2, _TCB, 64), lambda i: (0, i, 0)),
            pl.BlockSpec((_TCB, D_IN), lambda i: (i, 0)),
        ],
        out_shape=[
            jax.ShapeDtypeStruct((2, NP, 64), jnp.float32),
            jax.ShapeDtypeStruct((NP, D_IN), jnp.float32),
        ],
    )(deg_parts, x_p)


def _chain_body(s1p_ref, x_ref, dinvb_ref, W1_ref, b1_ref, W2_ref,
                table2_ref, P_ref):
    db = dinvb_ref[...]
    S1 = jnp.concatenate([s1p_ref[0], s1p_ref[1]], axis=1)
    agg1 = db * S1 + db * db * x_ref[...]
    h1 = jnp.maximum(
        jnp.dot(agg1, W1_ref[...], preferred_element_type=jnp.float32)
        + b1_ref[...], 0.0)
    P = jnp.dot(h1, W2_ref[...], preferred_element_type=jnp.float32)
    P_ref[...] = P
    table2_ref[...] = db[:, :48] * P


def _tc_chain(s1_parts, x_p, dinvb, W1, b1r, W2p):
    return pl.pallas_call(
        _chain_body,
        grid=(_GRID,),
        in_specs=[
            pl.BlockSpec((2, _TCB, 64), lambda i: (0, i, 0)),
            pl.BlockSpec((_TCB, D_IN), lambda i: (i, 0)),
            pl.BlockSpec((_TCB, D_IN), lambda i: (i, 0)),
            pl.BlockSpec((D_IN, D_HID), lambda i: (0, 0)),
            pl.BlockSpec((1, D_HID), lambda i: (0, 0)),
            pl.BlockSpec((D_HID, 48), lambda i: (0, 0)),
        ],
        out_specs=[
            pl.BlockSpec((_TCB, 48), lambda i: (i, 0)),
            pl.BlockSpec((_TCB, 48), lambda i: (i, 0)),
        ],
        out_shape=[
            jax.ShapeDtypeStruct((NP, 48), jnp.float32),
            jax.ShapeDtypeStruct((NP, 48), jnp.float32),
        ],
    )(s1_parts, x_p, dinvb, W1, b1r, W2p)


def _final_body(s2p_ref, P_ref, dinvb_ref, b2_ref, out_ref):
    db = dinvb_ref[:, :48]
    S2 = s2p_ref[0] + s2p_ref[1]
    P = P_ref[...]
    pre = db * S2 + db * db * P + b2_ref[...]
    mask = lax.broadcasted_iota(jnp.int32, (_TCB, 48), 1) < D_OUT
    neg = jnp.full_like(pre, -1e30)
    m = jnp.max(jnp.where(mask, pre, neg), axis=1, keepdims=True)
    e = jnp.where(mask, jnp.exp(pre - m), 0.0)
    ssum = jnp.sum(e, axis=1, keepdims=True)
    out_ref[...] = pre - m - jnp.log(ssum)


def _tc_final(s2_parts, P, dinvb, b2r):
    return pl.pallas_call(
        _final_body,
        grid=(_GRID,),
        in_specs=[
            pl.BlockSpec((2, _TCB, 48), lambda i: (0, i, 0)),
            pl.BlockSpec((_TCB, 48), lambda i: (i, 0)),
            pl.BlockSpec((_TCB, D_IN), lambda i: (i, 0)),
            pl.BlockSpec((1, 48), lambda i: (0, 0)),
        ],
        out_specs=pl.BlockSpec((_TCB, 48), lambda i: (i, 0)),
        out_shape=jax.ShapeDtypeStruct((NP, 48), jnp.float32),
    )(s2_parts, P, dinvb, b2r)


def kernel(x, edge_index, W1, b1, W2, b2):
    src = edge_index[0]
    dst = edge_index[1]
    pad = jnp.full((EP - N_EDGES,), N_NODES, dtype=jnp.int32)
    src_p = jnp.concatenate([src, pad]).reshape(EP // CH, CH)
    dst_p = jnp.concatenate([dst, pad]).reshape(EP // CH, CH)
    x_p = jnp.pad(x, ((0, NP - N_NODES), (0, 0)))
    W2p = jnp.pad(W2, ((0, 0), (0, 48 - D_OUT)))
    b1r = b1.reshape(1, D_HID)
    b2r = jnp.pad(b2, (0, 48 - D_OUT)).reshape(1, 48)

    ei_packed = jnp.stack([src_p, dst_p], axis=1)

    deg_parts = _sc_degree(dst_p)
    table1, dinvb = _tc_prep1(deg_parts, x_p)
    s1_parts = _sc_agg128(table1, ei_packed)
    table2, P = _tc_chain(s1_parts, x_p, dinvb, W1, b1r, W2p)
    s2_parts = _sc_agg48(table2, ei_packed)
    outp = _tc_final(s2_parts, P, dinvb, b2r)
    return outp[:N_NODES, :D_OUT]
